# baseline (speedup 1.0000x reference)
import functools

import jax
import jax.numpy as jnp
from jax import lax
from jax.experimental import pallas as pl
from jax.experimental.pallas import tpu as pltpu
from jax.experimental.pallas import tpu_sc as plsc

NN = 10000
NP = 10240
EE = 320000
EP = 320512
DD = 64
NCORE = 2
NSUB = 16
NW = NCORE * NSUB
EB = 3200
CB = 1280
NEGINF = float("-inf")


def _dg(a, b, ca, cb):
    return lax.dot_general(a, b, (((ca,), (cb,)), ((), ())),
                           preferred_element_type=jnp.float32)


def _relu(x):
    return jnp.maximum(x, 0.0)


def _mesh():
    return plsc.VectorSubcoreMesh(core_axis_name="c", subcore_axis_name="s",
                                  num_cores=NCORE, num_subcores=NSUB)


@functools.lru_cache(maxsize=None)
def _node_tables_call(transposed, interpret=False):
    def body(x_ref, lw, lb, sw, sb, dw, db, ta, tb):
        x = x_ref[...]
        if transposed:
            f = lambda w, b: _dg(x, w[...], 0, 0) + b[...]
        else:
            f = lambda w, b: _dg(x, w[...], 1, 0) + b[...]
        ta[...] = jnp.concatenate([f(sw, sb), f(lw, lb)], axis=1)
        tb[...] = jnp.concatenate(
            [f(dw, db), jnp.zeros((NP, DD), jnp.float32)], axis=1)

    osh = jax.ShapeDtypeStruct((NP, 2 * DD), jnp.float32)
    return pl.pallas_call(
        body,
        out_shape=[osh, osh],
        interpret=interpret,
    )


@functools.lru_cache(maxsize=None)
def _edge_call(interpret=False):
    def body(gD, gSX, pdx, pdy, pw1, pb1c, pw2, pb2r, pb2c,
             aw1, ab1c, aw2, ab2c, eyer, aTr, vTr):
        gad = gD[...][:, :DD]
        gas = gSX[...][:, :DD]
        gxv = gSX[...][:, DD:]
        pdr = jnp.concatenate([pdx[...], pdy[...]], axis=0)
        h1pT = _relu(_dg(pw1[...], pdr, 0, 0) + pb1c[...])
        delta = _relu(_dg(h1pT, pw2[...], 0, 0) + pb2r[...])
        deltaT = _relu(_dg(pw2[...], h1pT, 0, 0) + pb2c[...])
        apre = gad - gas + delta
        h1aT = _relu(_dg(aw1[...], apre, 0, 1) + ab1c[...])
        aTr[...] = _relu(_dg(aw2[...], h1aT, 0, 0) + ab2c[...])
        vTr[...] = _dg(eyer[...], gxv, 0, 1) + deltaT

    espec = pl.BlockSpec((EB, 2 * DD), lambda j: (j, 0))
    pspec = pl.BlockSpec((1, EB), lambda j: (0, j))
    w2d = pl.BlockSpec((2, DD), lambda j: (0, 0))
    wdd = pl.BlockSpec((DD, DD), lambda j: (0, 0))
    brow = pl.BlockSpec((1, DD), lambda j: (0, 0))
    bcol = pl.BlockSpec((DD, 1), lambda j: (0, 0))
    otsp = pl.BlockSpec((DD, EB), lambda j: (0, j))
    osh = jax.ShapeDtypeStruct((DD, EE), jnp.float32)
    return pl.pallas_call(
        body,
        grid=(EE // EB,),
        in_specs=[espec, espec, pspec, pspec,
                  w2d, bcol, wdd, brow, bcol,
                  wdd, bcol, wdd, bcol, wdd],
        out_specs=[otsp, otsp],
        out_shape=[osh, osh],
        interpret=interpret,
    )


@functools.lru_cache(maxsize=None)
def _pool_call(interpret=False):
    G = 16

    def body(xT_ref, b_ref, ow_ref, ob_ref, out_ref):
        xT = xT_ref[...]
        b = b_ref[0, :]
        cols = []
        for g in range(G):
            mg = jnp.where((b == g)[None, :], xT, NEGINF)
            cols.append(jnp.max(mg, axis=1))
        a = jnp.stack(cols, axis=1)
        a = jnp.where((a * 0.0) == 0.0, a, 0.0)
        out_ref[...] = _dg(a, ow_ref[...], 0, 0) + ob_ref[...]

    return pl.pallas_call(
        body,
        out_shape=jax.ShapeDtypeStruct((G, 2), jnp.float32),
        interpret=interpret,
    )


@functools.lru_cache(maxsize=None)
def _posdiff_call(interpret=False):
    nchunk = EE // CB
    per_w = (nchunk + NW - 1) // NW

    psh = jax.ShapeDtypeStruct((EE,), jnp.float32)

    @functools.partial(
        pl.kernel,
        out_type=(psh, psh),
        mesh=_mesh(),
        scratch_types=[
            pltpu.VMEM((NN,), jnp.float32),
            pltpu.VMEM((NN,), jnp.float32),
            pltpu.VMEM((CB,), jnp.int32),
            pltpu.VMEM((CB,), jnp.int32),
            pltpu.VMEM((CB,), jnp.float32),
            pltpu.VMEM((CB,), jnp.float32),
        ],
        compiler_params=pltpu.CompilerParams(needs_layout_passes=False),
        interpret=interpret,
    )
    def k(posx_h, posy_h, srcv, dstv, pdx, pdy, posx, posy, sbuf, dbuf, pxb, pyb):
        wid = lax.axis_index("s") * NCORE + lax.axis_index("c")
        pltpu.sync_copy(posx_h, posx)
        pltpu.sync_copy(posy_h, posy)

        def chunk_body(i, _):
            c = wid + NW * i

            @pl.when(c < nchunk)
            def _():
                base = c * CB
                pltpu.sync_copy(srcv.at[pl.ds(base, CB)], sbuf)
                pltpu.sync_copy(dstv.at[pl.ds(base, CB)], dbuf)

                def grp(j, _2):
                    sl = pl.ds(j * 16, 16)
                    s16 = sbuf[sl]
                    d16 = dbuf[sl]
                    pxb[sl] = (plsc.load_gather(posx, [d16])
                               - plsc.load_gather(posx, [s16]))
                    pyb[sl] = (plsc.load_gather(posy, [d16])
                               - plsc.load_gather(posy, [s16]))
                    return 0

                lax.fori_loop(0, CB // 16, grp, 0)
                pltpu.sync_copy(pxb, pdx.at[pl.ds(base, CB)])
                pltpu.sync_copy(pyb, pdy.at[pl.ds(base, CB)])

            return 0

        lax.fori_loop(0, per_w, chunk_body, 0)

    return k


@functools.lru_cache(maxsize=None)
def _gather3_call(interpret=False):
    nblk_total = EP // 1024
    per_w = (nblk_total + NW - 1) // NW
    osh = jax.ShapeDtypeStruct((EP, 2 * DD), jnp.float32)

    @functools.partial(
        pl.kernel,
        out_type=(osh, osh),
        mesh=_mesh(),
        scratch_types=[
            pltpu.VMEM((8, 128), jnp.int32),
            pltpu.VMEM((8, 128), jnp.int32),
            pltpu.VMEM((512, 2 * DD), jnp.float32),
            pltpu.SemaphoreType.DMA,
        ],
        interpret=interpret,
    )
    def k(ta, tb, src3, dst3, gsx, gd, sbuf, dbuf, buf, sem):
        wid = lax.axis_index("s") * NCORE + lax.axis_index("c")

        def blk_body(i, _):
            blk = wid + NW * i

            @pl.when(blk < nblk_total)
            def _():
                pltpu.sync_copy(src3.at[blk], sbuf)
                pltpu.sync_copy(dst3.at[blk], dbuf)
                for tbl, ibuf, out in ((tb, dbuf, gd), (ta, sbuf, gsx)):
                    for h in range(2):
                        descs = []
                        for j in range(4):
                            sl = pl.ds(j * 128, 128)
                            descs.append(pltpu.async_copy(
                                tbl.at[ibuf.at[h * 4 + j]], buf.at[sl], sem))
                        for dsc in descs:
                            dsc.wait()
                        e0 = blk * 1024 + h * 512
                        pltpu.sync_copy(buf, out.at[pl.ds(e0, 512)])

            return 0

        lax.fori_loop(0, per_w, blk_body, 0)

    return k


@functools.lru_cache(maxsize=None)
def _segment_call(interpret=False):
    @functools.partial(
        pl.kernel,
        out_type=jax.ShapeDtypeStruct((DD * NP,), jnp.float32),
        mesh=_mesh(),
        scratch_types=[
            pltpu.VMEM((NP,), jnp.float32),
            pltpu.VMEM((NP,), jnp.float32),
            pltpu.VMEM((NP,), jnp.float32),
            pltpu.VMEM((NP,), jnp.float32),
            pltpu.VMEM((NP,), jnp.float32),
            pltpu.VMEM((NP,), jnp.float32),
            pltpu.VMEM((NP,), jnp.int32),
            pltpu.VMEM((2, CB), jnp.int32),
            pltpu.VMEM((2, CB), jnp.float32),
            pltpu.VMEM((2, CB), jnp.float32),
            pltpu.VMEM((2, CB), jnp.float32),
            pltpu.VMEM((2, CB), jnp.float32),
            pltpu.VMEM((16,), jnp.float32),
            pltpu.VMEM((16,), jnp.int32),
            pltpu.SemaphoreType.DMA,
            pltpu.SemaphoreType.DMA,
        ],
        compiler_params=pltpu.CompilerParams(needs_layout_passes=False),
        interpret=interpret,
    )
    def k(aT, vT, dstv, zeros_c, neginf_c, xoutT,
          m0, m1, s0, s1, t0, t1, di, dbuf, a0b, a1b, v0b, v1b, lf, li,
          semA, semB):
        wid = lax.axis_index("s") * NCORE + lax.axis_index("c")
        c0 = 2 * wid
        c1 = c0 + 1
        iota = lax.iota(jnp.int32, 16)
        sems = (semA, semB)

        pltpu.sync_copy(neginf_c, m0)
        pltpu.sync_copy(neginf_c, m1)
        pltpu.sync_copy(neginf_c, t0)
        pltpu.sync_copy(neginf_c, t1)
        pltpu.sync_copy(zeros_c, s0)
        pltpu.sync_copy(zeros_c, s1)

        def permute(vals, pv):
            lf[...] = vals
            return plsc.load_gather(lf, [pv])

        def shifts_of_keys(sk):
            li[...] = sk
            takes = []
            for kk in (1, 2, 4, 8):
                skk = plsc.load_gather(li, [jnp.maximum(iota - kk, 0)])
                takes.append((sk == skk) & (iota >= kk))
            sku = plsc.load_gather(li, [jnp.minimum(iota + 1, 15)])
            ml = (sk != sku) | (iota == 15)
            return takes, ml

        def seg_scan(vals, takes, is_sum):
            v = vals
            for kk, take in zip((1, 2, 4, 8), takes):
                lf[...] = v
                sh = plsc.load_gather(lf, [jnp.maximum(iota - kk, 0)])
                if is_sum:
                    v = v + jnp.where(take, sh, 0.0)
                else:
                    v = jnp.where(take, jnp.maximum(v, sh), v)
            return v

        def rmw_max(acc, idxv, vals, mask=None):
            cur = plsc.load_gather(acc, [idxv], mask=mask)
            plsc.store_scatter(acc, [idxv], jnp.maximum(cur, vals), mask=mask)

        def rmw_add(acc, idxv, vals, mask=None):
            cur = plsc.load_gather(acc, [idxv], mask=mask)
            plsc.store_scatter(acc, [idxv], cur + vals, mask=mask)

        nch = EE // CB

        def detect(d16):
            plsc.store_scatter(di, [d16], iota)
            rb = plsc.load_gather(di, [d16])
            return rb != iota

        def p1_copies(slot, i):
            base = i * CB
            return [
                pltpu.make_async_copy(dstv.at[pl.ds(base, CB)],
                                      dbuf.at[slot], sems[slot]),
                pltpu.make_async_copy(
                    aT.at[pl.ds(pl.multiple_of(c0 * EE + base, 128), CB)],
                    a0b.at[slot], sems[slot]),
                pltpu.make_async_copy(
                    aT.at[pl.ds(pl.multiple_of(c1 * EE + base, 128), CB)],
                    a1b.at[slot], sems[slot]),
            ]

        def p1_start(slot, i):
            for d in p1_copies(slot, i):
                d.start()

        def p1_wait(slot, i):
            for d in p1_copies(slot, i):
                d.wait()

        def p1_proc(slot):
            def batch(jb, _):
                q0 = jb * 8
                mism = None
                for u in range(8):
                    d16 = dbuf[slot, pl.ds((q0 + u) * 16, 16)]
                    mm = detect(d16)
                    mism = mm if mism is None else (mism | mm)
                pred = jnp.any(mism)

                def slow():
                    for u in range(8):
                        sl = pl.ds((q0 + u) * 16, 16)
                        sk, pv = plsc.sort_key_val(dbuf[slot, sl], iota)
                        takes, ml = shifts_of_keys(sk)
                        rmw_max(m0, sk, seg_scan(permute(a0b[slot, sl], pv), takes, False), mask=ml)
                        rmw_max(m1, sk, seg_scan(permute(a1b[slot, sl], pv), takes, False), mask=ml)

                def fast():
                    for u in range(8):
                        sl = pl.ds((q0 + u) * 16, 16)
                        d16 = dbuf[slot, sl]
                        rmw_max(m0, d16, a0b[slot, sl])
                        rmw_max(m1, d16, a1b[slot, sl])

                lax.cond(pred, slow, fast)
                return 0

            lax.fori_loop(0, CB // 128, batch, 0)

        p1_start(0, 0)

        def p1_pair(ip, _):
            i0 = 2 * ip
            p1_wait(0, i0)
            p1_start(1, i0 + 1)
            p1_proc(0)
            p1_wait(1, i0 + 1)

            @pl.when(i0 + 2 < nch)
            def _():
                p1_start(0, i0 + 2)

            p1_proc(1)
            return 0

        lax.fori_loop(0, nch // 2, p1_pair, 0)

        def p2_copies(slot, i):
            base = i * CB
            return [
                pltpu.make_async_copy(dstv.at[pl.ds(base, CB)],
                                      dbuf.at[slot], sems[slot]),
                pltpu.make_async_copy(
                    aT.at[pl.ds(pl.multiple_of(c0 * EE + base, 128), CB)],
                    a0b.at[slot], sems[slot]),
                pltpu.make_async_copy(
                    aT.at[pl.ds(pl.multiple_of(c1 * EE + base, 128), CB)],
                    a1b.at[slot], sems[slot]),
                pltpu.make_async_copy(
                    vT.at[pl.ds(pl.multiple_of(c0 * EE + base, 128), CB)],
                    v0b.at[slot], sems[slot]),
                pltpu.make_async_copy(
                    vT.at[pl.ds(pl.multiple_of(c1 * EE + base, 128), CB)],
                    v1b.at[slot], sems[slot]),
            ]

        def p2_start(slot, i):
            for d in p2_copies(slot, i):
                d.start()

        def p2_wait(slot, i):
            for d in p2_copies(slot, i):
                d.wait()

        def p2_proc(slot):
            def batch(jb, _):
                q0 = jb * 8
                mism = None
                for u in range(8):
                    d16 = dbuf[slot, pl.ds((q0 + u) * 16, 16)]
                    mm = detect(d16)
                    mism = mm if mism is None else (mism | mm)
                pred = jnp.any(mism)

                def slow():
                    for u in range(8):
                        sl = pl.ds((q0 + u) * 16, 16)
                        d16 = dbuf[slot, sl]
                        e0 = jnp.exp(a0b[slot, sl] - plsc.load_gather(m0, [d16]))
                        e1 = jnp.exp(a1b[slot, sl] - plsc.load_gather(m1, [d16]))
                        p0 = e0 * v0b[slot, sl]
                        p1 = e1 * v1b[slot, sl]
                        sk, pv = plsc.sort_key_val(d16, iota)
                        takes, ml = shifts_of_keys(sk)
                        rmw_add(s0, sk, seg_scan(permute(e0, pv), takes, True), mask=ml)
                        rmw_add(s1, sk, seg_scan(permute(e1, pv), takes, True), mask=ml)
                        rmw_max(t0, sk, seg_scan(permute(p0, pv), takes, False), mask=ml)
                        rmw_max(t1, sk, seg_scan(permute(p1, pv), takes, False), mask=ml)

                def fast():
                    for u in range(8):
                        sl = pl.ds((q0 + u) * 16, 16)
                        d16 = dbuf[slot, sl]
                        e0 = jnp.exp(a0b[slot, sl] - plsc.load_gather(m0, [d16]))
                        e1 = jnp.exp(a1b[slot, sl] - plsc.load_gather(m1, [d16]))
                        plsc.addupdate_scatter(s0, [d16], e0)
                        plsc.addupdate_scatter(s1, [d16], e1)
                        rmw_max(t0, d16, e0 * v0b[slot, sl])
                        rmw_max(t1, d16, e1 * v1b[slot, sl])

                lax.cond(pred, slow, fast)
                return 0

            lax.fori_loop(0, CB // 128, batch, 0)

        p2_start(0, 0)

        def p2_pair(ip, _):
            i0 = 2 * ip
            p2_wait(0, i0)
            p2_start(1, i0 + 1)
            p2_proc(0)
            p2_wait(1, i0 + 1)

            @pl.when(i0 + 2 < nch)
            def _():
                p2_start(0, i0 + 2)

            p2_proc(1)
            return 0

        lax.fori_loop(0, nch // 2, p2_pair, 0)

        def ep_blk(i, _):
            base = i * CB

            def grp(j, _2):
                sl = pl.ds(base + j * 16, 16)
                osl = pl.ds(j * 16, 16)
                tv0 = t0[sl]
                tv1 = t1[sl]
                sv0 = s0[sl]
                sv1 = s1[sl]
                a0b[0, osl] = jnp.where((tv0 * 0.0) == 0.0, tv0 / (sv0 + 1e-16), 0.0)
                a1b[0, osl] = jnp.where((tv1 * 0.0) == 0.0, tv1 / (sv1 + 1e-16), 0.0)
                return 0

            lax.fori_loop(0, CB // 16, grp, 0)
            pltpu.sync_copy(a0b.at[0], xoutT.at[pl.ds(pl.multiple_of(c0 * NP + base, 128), CB)])
            pltpu.sync_copy(a1b.at[0], xoutT.at[pl.ds(pl.multiple_of(c1 * NP + base, 128), CB)])
            return 0

        lax.fori_loop(0, NP // CB, ep_blk, 0)

    return k


def _run(x_clusters, pos_clusters, edge_index_clusters, batch,
         lin_w, lin_b, src_w, src_b, dst_w, dst_b,
         pos_w1, pos_b1, pos_w2, pos_b2,
         attn_w1, attn_b1, attn_w2, attn_b2,
         out_w, out_b, interpret=False):
    f32 = jnp.float32
    src = edge_index_clusters[0]
    dst = edge_index_clusters[1]
    pad_e = EP - EE
    src3 = jnp.pad(src, (0, pad_e)).reshape(EP // 1024, 8, 128)
    dst3 = jnp.pad(dst, (0, pad_e)).reshape(EP // 1024, 8, 128)
    posx = pos_clusters[:, 0] + 0.0
    posy = pos_clusters[:, 1] + 0.0
    eye = jnp.eye(DD, dtype=f32)
    zeros_c = jnp.zeros((NP,), f32)
    neginf_c = jnp.full((NP,), NEGINF, f32)
    batch_p = jnp.pad(batch, (0, NP - NN), constant_values=16).reshape(1, NP)

    pdx, pdy = _posdiff_call(interpret)(posx, posy, src, dst)
    pdx = pdx.reshape(1, EE)
    pdy = pdy.reshape(1, EE)

    xt = jnp.pad(x_clusters, ((0, NP - NN), (0, 0)))
    nlayers = lin_w.shape[0]
    for i in range(nlayers):
        ta, tb = _node_tables_call(i > 0, interpret)(
            xt, lin_w[i], lin_b[i].reshape(1, DD),
            src_w[i], src_b[i].reshape(1, DD),
            dst_w[i], dst_b[i].reshape(1, DD))
        gsx, gd = _gather3_call(interpret)(ta, tb, src3, dst3)
        aT, vT = _edge_call(interpret)(
            gd, gsx, pdx, pdy,
            pos_w1[i], pos_b1[i].reshape(DD, 1),
            pos_w2[i], pos_b2[i].reshape(1, DD), pos_b2[i].reshape(DD, 1),
            attn_w1[i], attn_b1[i].reshape(DD, 1),
            attn_w2[i], attn_b2[i].reshape(DD, 1),
            eye)
        xt1 = _segment_call(interpret)(aT.reshape(DD * EE), vT.reshape(DD * EE),
                                       dst, zeros_c, neginf_c)
        xt = xt1.reshape(DD, NP)

    return _pool_call(interpret)(xt, batch_p, out_w, out_b.reshape(1, 2))


@jax.jit
def kernel(x_clusters, pos_clusters, edge_index_clusters, batch, add_cluster_pos,
           lin_w, lin_b, src_w, src_b, dst_w, dst_b,
           pos_w1, pos_b1, pos_w2, pos_b2,
           attn_w1, attn_b1, attn_w2, attn_b2,
           out_w, out_b):
    del add_cluster_pos
    return _run(x_clusters, pos_clusters, edge_index_clusters, batch,
                lin_w, lin_b, src_w, src_b, dst_w, dst_b,
                pos_w1, pos_b1, pos_w2, pos_b2,
                attn_w1, attn_b1, attn_w2, attn_b2,
                out_w, out_b)


# speedup vs baseline: 1.2632x; 1.2632x over previous
import functools

import jax
import jax.numpy as jnp
from jax import lax
from jax.experimental import pallas as pl
from jax.experimental.pallas import tpu as pltpu
from jax.experimental.pallas import tpu_sc as plsc

NN = 10000
NP = 10240
EE = 320000
EP = 320512
DD = 64
NCORE = 2
NSUB = 16
NW = NCORE * NSUB
EB = 3200
CB = 1280
NEGINF = float("-inf")


def _dg(a, b, ca, cb):
    return lax.dot_general(a, b, (((ca,), (cb,)), ((), ())),
                           preferred_element_type=jnp.float32)


def _relu(x):
    return jnp.maximum(x, 0.0)


def _mesh():
    return plsc.VectorSubcoreMesh(core_axis_name="c", subcore_axis_name="s",
                                  num_cores=NCORE, num_subcores=NSUB)


@functools.lru_cache(maxsize=None)
def _node_tables_call(transposed, interpret=False):
    def body(x_ref, lw, lb, sw, sb, dw, db, ta, tb):
        x = x_ref[...]
        if transposed:
            f = lambda w, b: _dg(x, w[...], 0, 0) + b[...]
        else:
            f = lambda w, b: _dg(x, w[...], 1, 0) + b[...]
        ta[...] = jnp.concatenate([f(sw, sb), f(lw, lb)], axis=1)
        tb[...] = jnp.concatenate(
            [f(dw, db), jnp.zeros((NP, DD), jnp.float32)], axis=1)

    osh = jax.ShapeDtypeStruct((NP, 2 * DD), jnp.float32)
    return pl.pallas_call(
        body,
        out_shape=[osh, osh],
        interpret=interpret,
    )


@functools.lru_cache(maxsize=None)
def _edge_call(interpret=False):
    def body(gD, gSX, pdx, pdy, pw1, pb1c, pw2, pb2r, pb2c,
             aw1, ab1c, aw2, ab2c, eyer, aTr, vTr):
        gad = gD[...][:, :DD]
        gas = gSX[...][:, :DD]
        gxv = gSX[...][:, DD:]
        pdr = jnp.concatenate([pdx[...], pdy[...]], axis=0)
        h1pT = _relu(_dg(pw1[...], pdr, 0, 0) + pb1c[...])
        delta = _relu(_dg(h1pT, pw2[...], 0, 0) + pb2r[...])
        deltaT = _relu(_dg(pw2[...], h1pT, 0, 0) + pb2c[...])
        apre = gad - gas + delta
        h1aT = _relu(_dg(aw1[...], apre, 0, 1) + ab1c[...])
        aTr[...] = _relu(_dg(aw2[...], h1aT, 0, 0) + ab2c[...])
        vTr[...] = _dg(eyer[...], gxv, 0, 1) + deltaT

    espec = pl.BlockSpec((EB, 2 * DD), lambda j: (j, 0))
    pspec = pl.BlockSpec((1, EB), lambda j: (0, j))
    w2d = pl.BlockSpec((2, DD), lambda j: (0, 0))
    wdd = pl.BlockSpec((DD, DD), lambda j: (0, 0))
    brow = pl.BlockSpec((1, DD), lambda j: (0, 0))
    bcol = pl.BlockSpec((DD, 1), lambda j: (0, 0))
    otsp = pl.BlockSpec((DD, EB), lambda j: (0, j))
    osh = jax.ShapeDtypeStruct((DD, EE), jnp.float32)
    return pl.pallas_call(
        body,
        grid=(EE // EB,),
        in_specs=[espec, espec, pspec, pspec,
                  w2d, bcol, wdd, brow, bcol,
                  wdd, bcol, wdd, bcol, wdd],
        out_specs=[otsp, otsp],
        out_shape=[osh, osh],
        interpret=interpret,
    )


@functools.lru_cache(maxsize=None)
def _pool_call(interpret=False):
    G = 16

    def body(xT_ref, b_ref, ow_ref, ob_ref, out_ref):
        xT = xT_ref[...]
        b = b_ref[0, :]
        cols = []
        for g in range(G):
            mg = jnp.where((b == g)[None, :], xT, NEGINF)
            cols.append(jnp.max(mg, axis=1))
        a = jnp.stack(cols, axis=1)
        a = jnp.where((a * 0.0) == 0.0, a, 0.0)
        out_ref[...] = _dg(a, ow_ref[...], 0, 0) + ob_ref[...]

    return pl.pallas_call(
        body,
        out_shape=jax.ShapeDtypeStruct((G, 2), jnp.float32),
        interpret=interpret,
    )


@functools.lru_cache(maxsize=None)
def _posdiff_call(interpret=False):
    nchunk = EE // CB
    per_w = (nchunk + NW - 1) // NW

    psh = jax.ShapeDtypeStruct((EE,), jnp.float32)

    @functools.partial(
        pl.kernel,
        out_type=(psh, psh),
        mesh=_mesh(),
        scratch_types=[
            pltpu.VMEM((NN,), jnp.float32),
            pltpu.VMEM((NN,), jnp.float32),
            pltpu.VMEM((CB,), jnp.int32),
            pltpu.VMEM((CB,), jnp.int32),
            pltpu.VMEM((CB,), jnp.float32),
            pltpu.VMEM((CB,), jnp.float32),
        ],
        compiler_params=pltpu.CompilerParams(needs_layout_passes=False),
        interpret=interpret,
    )
    def k(posx_h, posy_h, srcv, dstv, pdx, pdy, posx, posy, sbuf, dbuf, pxb, pyb):
        wid = lax.axis_index("s") * NCORE + lax.axis_index("c")
        pltpu.sync_copy(posx_h, posx)
        pltpu.sync_copy(posy_h, posy)

        def chunk_body(i, _):
            c = wid + NW * i

            @pl.when(c < nchunk)
            def _():
                base = c * CB
                pltpu.sync_copy(srcv.at[pl.ds(base, CB)], sbuf)
                pltpu.sync_copy(dstv.at[pl.ds(base, CB)], dbuf)

                def grp(j, _2):
                    sl = pl.ds(j * 16, 16)
                    s16 = sbuf[sl]
                    d16 = dbuf[sl]
                    pxb[sl] = (plsc.load_gather(posx, [d16])
                               - plsc.load_gather(posx, [s16]))
                    pyb[sl] = (plsc.load_gather(posy, [d16])
                               - plsc.load_gather(posy, [s16]))
                    return 0

                lax.fori_loop(0, CB // 16, grp, 0)
                pltpu.sync_copy(pxb, pdx.at[pl.ds(base, CB)])
                pltpu.sync_copy(pyb, pdy.at[pl.ds(base, CB)])

            return 0

        lax.fori_loop(0, per_w, chunk_body, 0)

    return k


@functools.lru_cache(maxsize=None)
def _gather3_call(interpret=False):
    nblk_total = EP // 1024
    per_w = (nblk_total + NW - 1) // NW
    osh = jax.ShapeDtypeStruct((EP, 2 * DD), jnp.float32)

    @functools.partial(
        pl.kernel,
        out_type=(osh, osh),
        mesh=_mesh(),
        scratch_types=[
            pltpu.VMEM((8, 128), jnp.int32),
            pltpu.VMEM((8, 128), jnp.int32),
            pltpu.VMEM((512, 2 * DD), jnp.float32),
            pltpu.SemaphoreType.DMA,
        ],
        interpret=interpret,
    )
    def k(ta, tb, src3, dst3, gsx, gd, sbuf, dbuf, buf, sem):
        wid = lax.axis_index("s") * NCORE + lax.axis_index("c")

        def blk_body(i, _):
            blk = wid + NW * i

            @pl.when(blk < nblk_total)
            def _():
                pltpu.sync_copy(src3.at[blk], sbuf)
                pltpu.sync_copy(dst3.at[blk], dbuf)
                for tbl, ibuf, out in ((tb, dbuf, gd), (ta, sbuf, gsx)):
                    for h in range(2):
                        descs = []
                        for j in range(4):
                            sl = pl.ds(j * 128, 128)
                            descs.append(pltpu.async_copy(
                                tbl.at[ibuf.at[h * 4 + j]], buf.at[sl], sem))
                        for dsc in descs:
                            dsc.wait()
                        e0 = blk * 1024 + h * 512
                        pltpu.sync_copy(buf, out.at[pl.ds(e0, 512)])

            return 0

        lax.fori_loop(0, per_w, blk_body, 0)

    return k


@functools.lru_cache(maxsize=None)
def _segment_call(interpret=False):
    @functools.partial(
        pl.kernel,
        out_type=jax.ShapeDtypeStruct((DD * NP,), jnp.float32),
        mesh=_mesh(),
        scratch_types=[
            pltpu.VMEM((NP,), jnp.float32),
            pltpu.VMEM((NP,), jnp.float32),
            pltpu.VMEM((NP,), jnp.float32),
            pltpu.VMEM((NP,), jnp.float32),
            pltpu.VMEM((NP,), jnp.float32),
            pltpu.VMEM((NP,), jnp.float32),
            pltpu.VMEM((NP,), jnp.int32),
            pltpu.VMEM((2, CB), jnp.int32),
            pltpu.VMEM((2, CB), jnp.float32),
            pltpu.VMEM((2, CB), jnp.float32),
            pltpu.VMEM((2, CB), jnp.float32),
            pltpu.VMEM((2, CB), jnp.float32),
            pltpu.VMEM((16,), jnp.float32),
            pltpu.VMEM((16,), jnp.int32),
            pltpu.SemaphoreType.DMA,
            pltpu.SemaphoreType.DMA,
        ],
        compiler_params=pltpu.CompilerParams(needs_layout_passes=False),
        interpret=interpret,
    )
    def k(aT, vT, dstv, zeros_c, neginf_c, xoutT,
          m0, m1, s0, s1, t0, t1, di, dbuf, a0b, a1b, v0b, v1b, lf, li,
          semA, semB):
        wid = lax.axis_index("s") * NCORE + lax.axis_index("c")
        c0 = 2 * wid
        c1 = c0 + 1
        iota = lax.iota(jnp.int32, 16)
        sems = (semA, semB)

        pltpu.sync_copy(neginf_c, m0)
        pltpu.sync_copy(neginf_c, m1)
        pltpu.sync_copy(neginf_c, t0)
        pltpu.sync_copy(neginf_c, t1)
        pltpu.sync_copy(zeros_c, s0)
        pltpu.sync_copy(zeros_c, s1)

        def permute(vals, pv):
            lf[...] = vals
            return plsc.load_gather(lf, [pv])

        def shifts_of_keys(sk):
            li[...] = sk
            takes = []
            for kk in (1, 2, 4, 8):
                skk = plsc.load_gather(li, [jnp.maximum(iota - kk, 0)])
                takes.append((sk == skk) & (iota >= kk))
            sku = plsc.load_gather(li, [jnp.minimum(iota + 1, 15)])
            ml = (sk != sku) | (iota == 15)
            return takes, ml

        def seg_scan(vals, takes, is_sum):
            v = vals
            for kk, take in zip((1, 2, 4, 8), takes):
                lf[...] = v
                sh = plsc.load_gather(lf, [jnp.maximum(iota - kk, 0)])
                if is_sum:
                    v = v + jnp.where(take, sh, 0.0)
                else:
                    v = jnp.where(take, jnp.maximum(v, sh), v)
            return v

        def rmw_max(acc, idxv, vals, mask=None):
            cur = plsc.load_gather(acc, [idxv], mask=mask)
            plsc.store_scatter(acc, [idxv], jnp.maximum(cur, vals), mask=mask)

        def rmw_add(acc, idxv, vals, mask=None):
            cur = plsc.load_gather(acc, [idxv], mask=mask)
            plsc.store_scatter(acc, [idxv], cur + vals, mask=mask)

        nch = EE // CB

        def detect(d16):
            plsc.store_scatter(di, [d16], iota)
            rb = plsc.load_gather(di, [d16])
            return rb != iota

        def p1_copies(slot, i):
            base = i * CB
            return [
                pltpu.make_async_copy(dstv.at[pl.ds(base, CB)],
                                      dbuf.at[slot], sems[slot]),
                pltpu.make_async_copy(
                    aT.at[pl.ds(pl.multiple_of(c0 * EE + base, 128), CB)],
                    a0b.at[slot], sems[slot]),
                pltpu.make_async_copy(
                    aT.at[pl.ds(pl.multiple_of(c1 * EE + base, 128), CB)],
                    a1b.at[slot], sems[slot]),
            ]

        def p1_start(slot, i):
            for d in p1_copies(slot, i):
                d.start()

        def p1_wait(slot, i):
            for d in p1_copies(slot, i):
                d.wait()

        def p1_proc(slot):
            def batch(jb, _):
                q0 = jb * 8
                d16s, a0s, a1s = [], [], []
                mism = None
                for u in range(8):
                    sl = pl.ds((q0 + u) * 16, 16)
                    d16 = dbuf[slot, sl]
                    mm = detect(d16)
                    mism = mm if mism is None else (mism | mm)
                    d16s.append(d16)
                    a0s.append(a0b[slot, sl])
                    a1s.append(a1b[slot, sl])

                def slow():
                    for u in range(8):
                        sk, pv = plsc.sort_key_val(d16s[u], iota)
                        takes, ml = shifts_of_keys(sk)
                        rmw_max(m0, sk, seg_scan(permute(a0s[u], pv), takes, False), mask=ml)
                        rmw_max(m1, sk, seg_scan(permute(a1s[u], pv), takes, False), mask=ml)

                def fast():
                    for u in range(8):
                        rmw_max(m0, d16s[u], a0s[u])
                        rmw_max(m1, d16s[u], a1s[u])

                lax.cond(jnp.any(mism), slow, fast)
                return 0

            lax.fori_loop(0, CB // 128, batch, 0)

        p1_start(0, 0)

        def p1_pair(ip, _):
            i0 = 2 * ip
            p1_wait(0, i0)
            p1_start(1, i0 + 1)
            p1_proc(0)
            p1_wait(1, i0 + 1)

            @pl.when(i0 + 2 < nch)
            def _():
                p1_start(0, i0 + 2)

            p1_proc(1)
            return 0

        lax.fori_loop(0, nch // 2, p1_pair, 0)

        def p2_copies(slot, i):
            base = i * CB
            return [
                pltpu.make_async_copy(dstv.at[pl.ds(base, CB)],
                                      dbuf.at[slot], sems[slot]),
                pltpu.make_async_copy(
                    aT.at[pl.ds(pl.multiple_of(c0 * EE + base, 128), CB)],
                    a0b.at[slot], sems[slot]),
                pltpu.make_async_copy(
                    aT.at[pl.ds(pl.multiple_of(c1 * EE + base, 128), CB)],
                    a1b.at[slot], sems[slot]),
                pltpu.make_async_copy(
                    vT.at[pl.ds(pl.multiple_of(c0 * EE + base, 128), CB)],
                    v0b.at[slot], sems[slot]),
                pltpu.make_async_copy(
                    vT.at[pl.ds(pl.multiple_of(c1 * EE + base, 128), CB)],
                    v1b.at[slot], sems[slot]),
            ]

        def p2_start(slot, i):
            for d in p2_copies(slot, i):
                d.start()

        def p2_wait(slot, i):
            for d in p2_copies(slot, i):
                d.wait()

        def p2_proc(slot):
            def batch(jb, _):
                q0 = jb * 4
                d16s, e0s, e1s, p0s, p1s = [], [], [], [], []
                mism = None
                for u in range(4):
                    sl = pl.ds((q0 + u) * 16, 16)
                    d16 = dbuf[slot, sl]
                    mm = detect(d16)
                    mism = mm if mism is None else (mism | mm)
                    e0 = jnp.exp(a0b[slot, sl] - plsc.load_gather(m0, [d16]))
                    e1 = jnp.exp(a1b[slot, sl] - plsc.load_gather(m1, [d16]))
                    d16s.append(d16)
                    e0s.append(e0)
                    e1s.append(e1)
                    p0s.append(e0 * v0b[slot, sl])
                    p1s.append(e1 * v1b[slot, sl])

                def slow():
                    for u in range(4):
                        sk, pv = plsc.sort_key_val(d16s[u], iota)
                        takes, ml = shifts_of_keys(sk)
                        rmw_add(s0, sk, seg_scan(permute(e0s[u], pv), takes, True), mask=ml)
                        rmw_add(s1, sk, seg_scan(permute(e1s[u], pv), takes, True), mask=ml)
                        rmw_max(t0, sk, seg_scan(permute(p0s[u], pv), takes, False), mask=ml)
                        rmw_max(t1, sk, seg_scan(permute(p1s[u], pv), takes, False), mask=ml)

                def fast():
                    for u in range(4):
                        plsc.addupdate_scatter(s0, [d16s[u]], e0s[u])
                        plsc.addupdate_scatter(s1, [d16s[u]], e1s[u])
                        rmw_max(t0, d16s[u], p0s[u])
                        rmw_max(t1, d16s[u], p1s[u])

                lax.cond(jnp.any(mism), slow, fast)
                return 0

            lax.fori_loop(0, CB // 64, batch, 0)

        p2_start(0, 0)

        def p2_pair(ip, _):
            i0 = 2 * ip
            p2_wait(0, i0)
            p2_start(1, i0 + 1)
            p2_proc(0)
            p2_wait(1, i0 + 1)

            @pl.when(i0 + 2 < nch)
            def _():
                p2_start(0, i0 + 2)

            p2_proc(1)
            return 0

        lax.fori_loop(0, nch // 2, p2_pair, 0)

        def ep_blk(i, _):
            base = i * CB

            def grp(j, _2):
                sl = pl.ds(base + j * 16, 16)
                osl = pl.ds(j * 16, 16)
                tv0 = t0[sl]
                tv1 = t1[sl]
                sv0 = s0[sl]
                sv1 = s1[sl]
                a0b[0, osl] = jnp.where((tv0 * 0.0) == 0.0, tv0 / (sv0 + 1e-16), 0.0)
                a1b[0, osl] = jnp.where((tv1 * 0.0) == 0.0, tv1 / (sv1 + 1e-16), 0.0)
                return 0

            lax.fori_loop(0, CB // 16, grp, 0)
            pltpu.sync_copy(a0b.at[0], xoutT.at[pl.ds(pl.multiple_of(c0 * NP + base, 128), CB)])
            pltpu.sync_copy(a1b.at[0], xoutT.at[pl.ds(pl.multiple_of(c1 * NP + base, 128), CB)])
            return 0

        lax.fori_loop(0, NP // CB, ep_blk, 0)

    return k


def _run(x_clusters, pos_clusters, edge_index_clusters, batch,
         lin_w, lin_b, src_w, src_b, dst_w, dst_b,
         pos_w1, pos_b1, pos_w2, pos_b2,
         attn_w1, attn_b1, attn_w2, attn_b2,
         out_w, out_b, interpret=False):
    f32 = jnp.float32
    src = edge_index_clusters[0]
    dst = edge_index_clusters[1]
    pad_e = EP - EE
    src3 = jnp.pad(src, (0, pad_e)).reshape(EP // 1024, 8, 128)
    dst3 = jnp.pad(dst, (0, pad_e)).reshape(EP // 1024, 8, 128)
    posx = pos_clusters[:, 0] + 0.0
    posy = pos_clusters[:, 1] + 0.0
    eye = jnp.eye(DD, dtype=f32)
    zeros_c = jnp.zeros((NP,), f32)
    neginf_c = jnp.full((NP,), NEGINF, f32)
    batch_p = jnp.pad(batch, (0, NP - NN), constant_values=16).reshape(1, NP)

    pdx, pdy = _posdiff_call(interpret)(posx, posy, src, dst)
    pdx = pdx.reshape(1, EE)
    pdy = pdy.reshape(1, EE)

    xt = jnp.pad(x_clusters, ((0, NP - NN), (0, 0)))
    nlayers = lin_w.shape[0]
    for i in range(nlayers):
        ta, tb = _node_tables_call(i > 0, interpret)(
            xt, lin_w[i], lin_b[i].reshape(1, DD),
            src_w[i], src_b[i].reshape(1, DD),
            dst_w[i], dst_b[i].reshape(1, DD))
        gsx, gd = _gather3_call(interpret)(ta, tb, src3, dst3)
        aT, vT = _edge_call(interpret)(
            gd, gsx, pdx, pdy,
            pos_w1[i], pos_b1[i].reshape(DD, 1),
            pos_w2[i], pos_b2[i].reshape(1, DD), pos_b2[i].reshape(DD, 1),
            attn_w1[i], attn_b1[i].reshape(DD, 1),
            attn_w2[i], attn_b2[i].reshape(DD, 1),
            eye)
        xt1 = _segment_call(interpret)(aT.reshape(DD * EE), vT.reshape(DD * EE),
                                       dst, zeros_c, neginf_c)
        xt = xt1.reshape(DD, NP)

    return _pool_call(interpret)(xt, batch_p, out_w, out_b.reshape(1, 2))


@jax.jit
def kernel(x_clusters, pos_clusters, edge_index_clusters, batch, add_cluster_pos,
           lin_w, lin_b, src_w, src_b, dst_w, dst_b,
           pos_w1, pos_b1, pos_w2, pos_b2,
           attn_w1, attn_b1, attn_w2, attn_b2,
           out_w, out_b):
    del add_cluster_pos
    return _run(x_clusters, pos_clusters, edge_index_clusters, batch,
                lin_w, lin_b, src_w, src_b, dst_w, dst_b,
                pos_w1, pos_b1, pos_w2, pos_b2,
                attn_w1, attn_b1, attn_w2, attn_b2,
                out_w, out_b)


# speedup vs baseline: 1.2837x; 1.0163x over previous
import functools

import jax
import jax.numpy as jnp
from jax import lax
from jax.experimental import pallas as pl
from jax.experimental.pallas import tpu as pltpu
from jax.experimental.pallas import tpu_sc as plsc

NN = 10000
NP = 10240
EE = 320000
EP = 320512
DD = 64
NCORE = 2
NSUB = 16
NW = NCORE * NSUB
EB = 3200
CB = 1280
NEGINF = float("-inf")


def _dg(a, b, ca, cb):
    return lax.dot_general(a, b, (((ca,), (cb,)), ((), ())),
                           preferred_element_type=jnp.float32)


def _relu(x):
    return jnp.maximum(x, 0.0)


def _mesh():
    return plsc.VectorSubcoreMesh(core_axis_name="c", subcore_axis_name="s",
                                  num_cores=NCORE, num_subcores=NSUB)


@functools.lru_cache(maxsize=None)
def _node_tables_call(transposed, interpret=False):
    def body(x_ref, lw, lb, sw, sb, dw, db, ta, tb):
        x = x_ref[...]
        if transposed:
            f = lambda w, b: _dg(x, w[...], 0, 0) + b[...]
        else:
            f = lambda w, b: _dg(x, w[...], 1, 0) + b[...]
        ta[...] = jnp.concatenate([f(sw, sb), f(lw, lb)], axis=1)
        tb[...] = jnp.concatenate(
            [f(dw, db), jnp.zeros((NP, DD), jnp.float32)], axis=1)

    osh = jax.ShapeDtypeStruct((NP, 2 * DD), jnp.float32)
    return pl.pallas_call(
        body,
        out_shape=[osh, osh],
        interpret=interpret,
    )


@functools.lru_cache(maxsize=None)
def _edge_call(interpret=False):
    def body(gD, gSX, pdx, pdy, pw1, pb1c, pw2, pb2r, pb2c,
             aw1, ab1c, aw2, ab2c, eyer, aTr, vTr):
        gad = gD[...][:, :DD]
        gas = gSX[...][:, :DD]
        gxv = gSX[...][:, DD:]
        pdr = jnp.concatenate([pdx[...], pdy[...]], axis=0)
        h1pT = _relu(_dg(pw1[...], pdr, 0, 0) + pb1c[...])
        delta = _relu(_dg(h1pT, pw2[...], 0, 0) + pb2r[...])
        deltaT = _relu(_dg(pw2[...], h1pT, 0, 0) + pb2c[...])
        apre = gad - gas + delta
        h1aT = _relu(_dg(aw1[...], apre, 0, 1) + ab1c[...])
        aTr[...] = _relu(_dg(aw2[...], h1aT, 0, 0) + ab2c[...])
        vTr[...] = _dg(eyer[...], gxv, 0, 1) + deltaT

    espec = pl.BlockSpec((EB, 2 * DD), lambda j: (j, 0))
    pspec = pl.BlockSpec((1, EB), lambda j: (0, j))
    w2d = pl.BlockSpec((2, DD), lambda j: (0, 0))
    wdd = pl.BlockSpec((DD, DD), lambda j: (0, 0))
    brow = pl.BlockSpec((1, DD), lambda j: (0, 0))
    bcol = pl.BlockSpec((DD, 1), lambda j: (0, 0))
    otsp = pl.BlockSpec((DD, EB), lambda j: (0, j))
    osh = jax.ShapeDtypeStruct((DD, EE), jnp.float32)
    return pl.pallas_call(
        body,
        grid=(EE // EB,),
        in_specs=[espec, espec, pspec, pspec,
                  w2d, bcol, wdd, brow, bcol,
                  wdd, bcol, wdd, bcol, wdd],
        out_specs=[otsp, otsp],
        out_shape=[osh, osh],
        interpret=interpret,
    )


@functools.lru_cache(maxsize=None)
def _pool_call(interpret=False):
    G = 16

    def body(xT_ref, b_ref, ow_ref, ob_ref, out_ref):
        xT = xT_ref[...]
        b = b_ref[0, :]
        cols = []
        for g in range(G):
            mg = jnp.where((b == g)[None, :], xT, NEGINF)
            cols.append(jnp.max(mg, axis=1))
        a = jnp.stack(cols, axis=1)
        a = jnp.where((a * 0.0) == 0.0, a, 0.0)
        out_ref[...] = _dg(a, ow_ref[...], 0, 0) + ob_ref[...]

    return pl.pallas_call(
        body,
        out_shape=jax.ShapeDtypeStruct((G, 2), jnp.float32),
        interpret=interpret,
    )


@functools.lru_cache(maxsize=None)
def _posdiff_call(interpret=False):
    nchunk = EE // CB
    per_w = (nchunk + NW - 1) // NW

    psh = jax.ShapeDtypeStruct((EE,), jnp.float32)

    @functools.partial(
        pl.kernel,
        out_type=(psh, psh),
        mesh=_mesh(),
        scratch_types=[
            pltpu.VMEM((NN,), jnp.float32),
            pltpu.VMEM((NN,), jnp.float32),
            pltpu.VMEM((CB,), jnp.int32),
            pltpu.VMEM((CB,), jnp.int32),
            pltpu.VMEM((CB,), jnp.float32),
            pltpu.VMEM((CB,), jnp.float32),
        ],
        compiler_params=pltpu.CompilerParams(needs_layout_passes=False),
        interpret=interpret,
    )
    def k(posx_h, posy_h, srcv, dstv, pdx, pdy, posx, posy, sbuf, dbuf, pxb, pyb):
        wid = lax.axis_index("s") * NCORE + lax.axis_index("c")
        pltpu.sync_copy(posx_h, posx)
        pltpu.sync_copy(posy_h, posy)

        def chunk_body(i, _):
            c = wid + NW * i

            @pl.when(c < nchunk)
            def _():
                base = c * CB
                pltpu.sync_copy(srcv.at[pl.ds(base, CB)], sbuf)
                pltpu.sync_copy(dstv.at[pl.ds(base, CB)], dbuf)

                def grp(j, _2):
                    sl = pl.ds(j * 16, 16)
                    s16 = sbuf[sl]
                    d16 = dbuf[sl]
                    pxb[sl] = (plsc.load_gather(posx, [d16])
                               - plsc.load_gather(posx, [s16]))
                    pyb[sl] = (plsc.load_gather(posy, [d16])
                               - plsc.load_gather(posy, [s16]))
                    return 0

                lax.fori_loop(0, CB // 16, grp, 0)
                pltpu.sync_copy(pxb, pdx.at[pl.ds(base, CB)])
                pltpu.sync_copy(pyb, pdy.at[pl.ds(base, CB)])

            return 0

        lax.fori_loop(0, per_w, chunk_body, 0)

    return k


@functools.lru_cache(maxsize=None)
def _gather3_call(interpret=False):
    nblk_total = EP // 1024
    per_w = (nblk_total + NW - 1) // NW
    osh = jax.ShapeDtypeStruct((EP, 2 * DD), jnp.float32)

    @functools.partial(
        pl.kernel,
        out_type=(osh, osh),
        mesh=_mesh(),
        scratch_types=[
            pltpu.VMEM((8, 128), jnp.int32),
            pltpu.VMEM((8, 128), jnp.int32),
            pltpu.VMEM((3, 256, 2 * DD), jnp.float32),
            pltpu.SemaphoreType.DMA,
            pltpu.SemaphoreType.DMA,
            pltpu.SemaphoreType.DMA,
            pltpu.SemaphoreType.DMA,
            pltpu.SemaphoreType.DMA,
            pltpu.SemaphoreType.DMA,
        ],
        interpret=interpret,
    )
    def k(ta, tb, src3, dst3, gsx, gd, sbuf, dbuf, bufs,
          g0, g1, g2, o0, o1, o2):
        wid = lax.axis_index("s") * NCORE + lax.axis_index("c")
        gsem = (g0, g1, g2)
        osem = (o0, o1, o2)

        def blk_body(i, _):
            blk = wid + NW * i

            @pl.when(blk < nblk_total)
            def _():
                pltpu.sync_copy(src3.at[blk], sbuf)
                pltpu.sync_copy(dst3.at[blk], dbuf)
                units = []
                for un in range(8):
                    tbl, ibuf = (tb, dbuf) if un < 4 else (ta, sbuf)
                    q = un % 4
                    out = gd if un < 4 else gsx
                    e0 = blk * 1024 + q * 256
                    units.append((tbl, ibuf, q, out, e0))

                def gstart(un):
                    tbl, ibuf, q, out, e0 = units[un]
                    s = un % 3
                    for j in range(2):
                        pltpu.async_copy(tbl.at[ibuf.at[q * 2 + j]],
                                         bufs.at[s, pl.ds(j * 128, 128)],
                                         gsem[s])

                def gwait(un):
                    tbl, ibuf, q, out, e0 = units[un]
                    s = un % 3
                    for j in range(2):
                        pltpu.make_async_copy(
                            tbl.at[ibuf.at[q * 2 + j]],
                            bufs.at[s, pl.ds(j * 128, 128)], gsem[s]).wait()

                def ostart(un):
                    tbl, ibuf, q, out, e0 = units[un]
                    s = un % 3
                    pltpu.async_copy(bufs.at[s], out.at[pl.ds(e0, 256)], osem[s])

                def owait(un):
                    tbl, ibuf, q, out, e0 = units[un]
                    s = un % 3
                    pltpu.make_async_copy(bufs.at[s], out.at[pl.ds(e0, 256)],
                                          osem[s]).wait()

                gstart(0)
                gstart(1)
                for un in range(8):
                    gwait(un)
                    ostart(un)
                    if un + 2 < 8:
                        if un >= 1:
                            owait(un - 1)
                        gstart(un + 2)
                owait(6)
                owait(7)

            return 0

        lax.fori_loop(0, per_w, blk_body, 0)

    return k


@functools.lru_cache(maxsize=None)
def _segment_call(interpret=False):
    @functools.partial(
        pl.kernel,
        out_type=jax.ShapeDtypeStruct((DD * NP,), jnp.float32),
        mesh=_mesh(),
        scratch_types=[
            pltpu.VMEM((NP,), jnp.float32),
            pltpu.VMEM((NP,), jnp.float32),
            pltpu.VMEM((NP,), jnp.float32),
            pltpu.VMEM((NP,), jnp.float32),
            pltpu.VMEM((NP,), jnp.float32),
            pltpu.VMEM((NP,), jnp.float32),
            pltpu.VMEM((NP,), jnp.int32),
            pltpu.VMEM((2, CB), jnp.int32),
            pltpu.VMEM((2, CB), jnp.float32),
            pltpu.VMEM((2, CB), jnp.float32),
            pltpu.VMEM((2, CB), jnp.float32),
            pltpu.VMEM((2, CB), jnp.float32),
            pltpu.VMEM((16,), jnp.float32),
            pltpu.VMEM((16,), jnp.int32),
            pltpu.SemaphoreType.DMA,
            pltpu.SemaphoreType.DMA,
        ],
        compiler_params=pltpu.CompilerParams(needs_layout_passes=False),
        interpret=interpret,
    )
    def k(aT, vT, dstv, zeros_c, neginf_c, xoutT,
          m0, m1, s0, s1, t0, t1, di, dbuf, a0b, a1b, v0b, v1b, lf, li,
          semA, semB):
        wid = lax.axis_index("s") * NCORE + lax.axis_index("c")
        c0 = 2 * wid
        c1 = c0 + 1
        iota = lax.iota(jnp.int32, 16)
        sems = (semA, semB)

        pltpu.sync_copy(neginf_c, m0)
        pltpu.sync_copy(neginf_c, m1)
        pltpu.sync_copy(neginf_c, t0)
        pltpu.sync_copy(neginf_c, t1)
        pltpu.sync_copy(zeros_c, s0)
        pltpu.sync_copy(zeros_c, s1)

        def permute(vals, pv):
            lf[...] = vals
            return plsc.load_gather(lf, [pv])

        def shifts_of_keys(sk):
            li[...] = sk
            takes = []
            for kk in (1, 2, 4, 8):
                skk = plsc.load_gather(li, [jnp.maximum(iota - kk, 0)])
                takes.append((sk == skk) & (iota >= kk))
            sku = plsc.load_gather(li, [jnp.minimum(iota + 1, 15)])
            ml = (sk != sku) | (iota == 15)
            return takes, ml

        def seg_scan(vals, takes, is_sum):
            v = vals
            for kk, take in zip((1, 2, 4, 8), takes):
                lf[...] = v
                sh = plsc.load_gather(lf, [jnp.maximum(iota - kk, 0)])
                if is_sum:
                    v = v + jnp.where(take, sh, 0.0)
                else:
                    v = jnp.where(take, jnp.maximum(v, sh), v)
            return v

        def rmw_max(acc, idxv, vals, mask=None):
            cur = plsc.load_gather(acc, [idxv], mask=mask)
            plsc.store_scatter(acc, [idxv], jnp.maximum(cur, vals), mask=mask)

        def rmw_add(acc, idxv, vals, mask=None):
            cur = plsc.load_gather(acc, [idxv], mask=mask)
            plsc.store_scatter(acc, [idxv], cur + vals, mask=mask)

        nch = EE // CB

        def detect(d16):
            plsc.store_scatter(di, [d16], iota)
            rb = plsc.load_gather(di, [d16])
            return rb != iota

        def p1_copies(slot, i):
            base = i * CB
            return [
                pltpu.make_async_copy(dstv.at[pl.ds(base, CB)],
                                      dbuf.at[slot], sems[slot]),
                pltpu.make_async_copy(
                    aT.at[pl.ds(pl.multiple_of(c0 * EE + base, 128), CB)],
                    a0b.at[slot], sems[slot]),
                pltpu.make_async_copy(
                    aT.at[pl.ds(pl.multiple_of(c1 * EE + base, 128), CB)],
                    a1b.at[slot], sems[slot]),
            ]

        def p1_start(slot, i):
            for d in p1_copies(slot, i):
                d.start()

        def p1_wait(slot, i):
            for d in p1_copies(slot, i):
                d.wait()

        def p1_proc(slot):
            def batch(jb, _):
                q0 = jb * 8
                d16s, a0s, a1s = [], [], []
                mism = None
                for u in range(8):
                    sl = pl.ds((q0 + u) * 16, 16)
                    d16 = dbuf[slot, sl]
                    mm = detect(d16)
                    mism = mm if mism is None else (mism | mm)
                    d16s.append(d16)
                    a0s.append(a0b[slot, sl])
                    a1s.append(a1b[slot, sl])

                def slow():
                    for u in range(8):
                        sk, pv = plsc.sort_key_val(d16s[u], iota)
                        takes, ml = shifts_of_keys(sk)
                        rmw_max(m0, sk, seg_scan(permute(a0s[u], pv), takes, False), mask=ml)
                        rmw_max(m1, sk, seg_scan(permute(a1s[u], pv), takes, False), mask=ml)

                def fast():
                    for u in range(8):
                        rmw_max(m0, d16s[u], a0s[u])
                        rmw_max(m1, d16s[u], a1s[u])

                lax.cond(jnp.any(mism), slow, fast)
                return 0

            lax.fori_loop(0, CB // 128, batch, 0)

        p1_start(0, 0)

        def p1_pair(ip, _):
            i0 = 2 * ip
            p1_wait(0, i0)
            p1_start(1, i0 + 1)
            p1_proc(0)
            p1_wait(1, i0 + 1)

            @pl.when(i0 + 2 < nch)
            def _():
                p1_start(0, i0 + 2)

            p1_proc(1)
            return 0

        lax.fori_loop(0, nch // 2, p1_pair, 0)

        def p2_copies(slot, i):
            base = i * CB
            return [
                pltpu.make_async_copy(dstv.at[pl.ds(base, CB)],
                                      dbuf.at[slot], sems[slot]),
                pltpu.make_async_copy(
                    aT.at[pl.ds(pl.multiple_of(c0 * EE + base, 128), CB)],
                    a0b.at[slot], sems[slot]),
                pltpu.make_async_copy(
                    aT.at[pl.ds(pl.multiple_of(c1 * EE + base, 128), CB)],
                    a1b.at[slot], sems[slot]),
                pltpu.make_async_copy(
                    vT.at[pl.ds(pl.multiple_of(c0 * EE + base, 128), CB)],
                    v0b.at[slot], sems[slot]),
                pltpu.make_async_copy(
                    vT.at[pl.ds(pl.multiple_of(c1 * EE + base, 128), CB)],
                    v1b.at[slot], sems[slot]),
            ]

        def p2_start(slot, i):
            for d in p2_copies(slot, i):
                d.start()

        def p2_wait(slot, i):
            for d in p2_copies(slot, i):
                d.wait()

        def p2_proc(slot):
            def batch(jb, _):
                q0 = jb * 4
                d16s, e0s, e1s, p0s, p1s = [], [], [], [], []
                mism = None
                for u in range(4):
                    sl = pl.ds((q0 + u) * 16, 16)
                    d16 = dbuf[slot, sl]
                    mm = detect(d16)
                    mism = mm if mism is None else (mism | mm)
                    e0 = jnp.exp(a0b[slot, sl] - plsc.load_gather(m0, [d16]))
                    e1 = jnp.exp(a1b[slot, sl] - plsc.load_gather(m1, [d16]))
                    d16s.append(d16)
                    e0s.append(e0)
                    e1s.append(e1)
                    p0s.append(e0 * v0b[slot, sl])
                    p1s.append(e1 * v1b[slot, sl])

                def slow():
                    for u in range(4):
                        sk, pv = plsc.sort_key_val(d16s[u], iota)
                        takes, ml = shifts_of_keys(sk)
                        rmw_add(s0, sk, seg_scan(permute(e0s[u], pv), takes, True), mask=ml)
                        rmw_add(s1, sk, seg_scan(permute(e1s[u], pv), takes, True), mask=ml)
                        rmw_max(t0, sk, seg_scan(permute(p0s[u], pv), takes, False), mask=ml)
                        rmw_max(t1, sk, seg_scan(permute(p1s[u], pv), takes, False), mask=ml)

                def fast():
                    for u in range(4):
                        plsc.addupdate_scatter(s0, [d16s[u]], e0s[u])
                        plsc.addupdate_scatter(s1, [d16s[u]], e1s[u])
                        rmw_max(t0, d16s[u], p0s[u])
                        rmw_max(t1, d16s[u], p1s[u])

                lax.cond(jnp.any(mism), slow, fast)
                return 0

            lax.fori_loop(0, CB // 64, batch, 0)

        p2_start(0, 0)

        def p2_pair(ip, _):
            i0 = 2 * ip
            p2_wait(0, i0)
            p2_start(1, i0 + 1)
            p2_proc(0)
            p2_wait(1, i0 + 1)

            @pl.when(i0 + 2 < nch)
            def _():
                p2_start(0, i0 + 2)

            p2_proc(1)
            return 0

        lax.fori_loop(0, nch // 2, p2_pair, 0)

        def ep_blk(i, _):
            base = i * CB

            def grp(j, _2):
                sl = pl.ds(base + j * 16, 16)
                osl = pl.ds(j * 16, 16)
                tv0 = t0[sl]
                tv1 = t1[sl]
                sv0 = s0[sl]
                sv1 = s1[sl]
                a0b[0, osl] = jnp.where((tv0 * 0.0) == 0.0, tv0 / (sv0 + 1e-16), 0.0)
                a1b[0, osl] = jnp.where((tv1 * 0.0) == 0.0, tv1 / (sv1 + 1e-16), 0.0)
                return 0

            lax.fori_loop(0, CB // 16, grp, 0)
            pltpu.sync_copy(a0b.at[0], xoutT.at[pl.ds(pl.multiple_of(c0 * NP + base, 128), CB)])
            pltpu.sync_copy(a1b.at[0], xoutT.at[pl.ds(pl.multiple_of(c1 * NP + base, 128), CB)])
            return 0

        lax.fori_loop(0, NP // CB, ep_blk, 0)

    return k


def _run(x_clusters, pos_clusters, edge_index_clusters, batch,
         lin_w, lin_b, src_w, src_b, dst_w, dst_b,
         pos_w1, pos_b1, pos_w2, pos_b2,
         attn_w1, attn_b1, attn_w2, attn_b2,
         out_w, out_b, interpret=False):
    f32 = jnp.float32
    src = edge_index_clusters[0]
    dst = edge_index_clusters[1]
    pad_e = EP - EE
    src3 = jnp.pad(src, (0, pad_e)).reshape(EP // 1024, 8, 128)
    dst3 = jnp.pad(dst, (0, pad_e)).reshape(EP // 1024, 8, 128)
    posx = pos_clusters[:, 0] + 0.0
    posy = pos_clusters[:, 1] + 0.0
    eye = jnp.eye(DD, dtype=f32)
    zeros_c = jnp.zeros((NP,), f32)
    neginf_c = jnp.full((NP,), NEGINF, f32)
    batch_p = jnp.pad(batch, (0, NP - NN), constant_values=16).reshape(1, NP)

    pdx, pdy = _posdiff_call(interpret)(posx, posy, src, dst)
    pdx = pdx.reshape(1, EE)
    pdy = pdy.reshape(1, EE)

    xt = jnp.pad(x_clusters, ((0, NP - NN), (0, 0)))
    nlayers = lin_w.shape[0]
    for i in range(nlayers):
        ta, tb = _node_tables_call(i > 0, interpret)(
            xt, lin_w[i], lin_b[i].reshape(1, DD),
            src_w[i], src_b[i].reshape(1, DD),
            dst_w[i], dst_b[i].reshape(1, DD))
        gsx, gd = _gather3_call(interpret)(ta, tb, src3, dst3)
        aT, vT = _edge_call(interpret)(
            gd, gsx, pdx, pdy,
            pos_w1[i], pos_b1[i].reshape(DD, 1),
            pos_w2[i], pos_b2[i].reshape(1, DD), pos_b2[i].reshape(DD, 1),
            attn_w1[i], attn_b1[i].reshape(DD, 1),
            attn_w2[i], attn_b2[i].reshape(DD, 1),
            eye)
        xt1 = _segment_call(interpret)(aT.reshape(DD * EE), vT.reshape(DD * EE),
                                       dst, zeros_c, neginf_c)
        xt = xt1.reshape(DD, NP)

    return _pool_call(interpret)(xt, batch_p, out_w, out_b.reshape(1, 2))


@jax.jit
def kernel(x_clusters, pos_clusters, edge_index_clusters, batch, add_cluster_pos,
           lin_w, lin_b, src_w, src_b, dst_w, dst_b,
           pos_w1, pos_b1, pos_w2, pos_b2,
           attn_w1, attn_b1, attn_w2, attn_b2,
           out_w, out_b):
    del add_cluster_pos
    return _run(x_clusters, pos_clusters, edge_index_clusters, batch,
                lin_w, lin_b, src_w, src_b, dst_w, dst_b,
                pos_w1, pos_b1, pos_w2, pos_b2,
                attn_w1, attn_b1, attn_w2, attn_b2,
                out_w, out_b)


# speedup vs baseline: 1.2872x; 1.0027x over previous
import functools

import jax
import jax.numpy as jnp
from jax import lax
from jax.experimental import pallas as pl
from jax.experimental.pallas import tpu as pltpu
from jax.experimental.pallas import tpu_sc as plsc

NN = 10000
NP = 10240
EE = 320000
EP = 320512
DD = 64
NCORE = 2
NSUB = 16
NW = NCORE * NSUB
EB = 3200
CB = 1280
NEGINF = float("-inf")


def _dg(a, b, ca, cb):
    return lax.dot_general(a, b, (((ca,), (cb,)), ((), ())),
                           preferred_element_type=jnp.float32)


def _relu(x):
    return jnp.maximum(x, 0.0)


def _mesh():
    return plsc.VectorSubcoreMesh(core_axis_name="c", subcore_axis_name="s",
                                  num_cores=NCORE, num_subcores=NSUB)


@functools.lru_cache(maxsize=None)
def _node_tables_call(transposed, interpret=False):
    def body(x_ref, lw, lb, sw, sb, dw, db, ta, tb):
        x = x_ref[...]
        if transposed:
            f = lambda w, b: _dg(x, w[...], 0, 0) + b[...]
        else:
            f = lambda w, b: _dg(x, w[...], 1, 0) + b[...]
        ta[...] = jnp.concatenate([f(sw, sb), f(lw, lb)], axis=1)
        tb[...] = jnp.concatenate(
            [f(dw, db), jnp.zeros((NP, DD), jnp.float32)], axis=1)

    osh = jax.ShapeDtypeStruct((NP, 2 * DD), jnp.float32)
    return pl.pallas_call(
        body,
        out_shape=[osh, osh],
        interpret=interpret,
    )


@functools.lru_cache(maxsize=None)
def _edge_call(interpret=False):
    def body(gD, gSX, pdx, pdy, pw1, pb1c, pw2, pb2r, pb2c,
             aw1, ab1c, aw2, ab2c, eyer, aTr, vTr):
        gad = gD[...][:, :DD]
        gas = gSX[...][:, :DD]
        gxv = gSX[...][:, DD:]
        pdr = jnp.concatenate([pdx[...], pdy[...]], axis=0)
        h1pT = _relu(_dg(pw1[...], pdr, 0, 0) + pb1c[...])
        delta = _relu(_dg(h1pT, pw2[...], 0, 0) + pb2r[...])
        deltaT = _relu(_dg(pw2[...], h1pT, 0, 0) + pb2c[...])
        apre = gad - gas + delta
        h1aT = _relu(_dg(aw1[...], apre, 0, 1) + ab1c[...])
        aTr[...] = _relu(_dg(aw2[...], h1aT, 0, 0) + ab2c[...])
        vTr[...] = _dg(eyer[...], gxv, 0, 1) + deltaT

    espec = pl.BlockSpec((EB, 2 * DD), lambda j: (j, 0))
    pspec = pl.BlockSpec((1, EB), lambda j: (0, j))
    w2d = pl.BlockSpec((2, DD), lambda j: (0, 0))
    wdd = pl.BlockSpec((DD, DD), lambda j: (0, 0))
    brow = pl.BlockSpec((1, DD), lambda j: (0, 0))
    bcol = pl.BlockSpec((DD, 1), lambda j: (0, 0))
    otsp = pl.BlockSpec((DD, EB), lambda j: (0, j))
    osh = jax.ShapeDtypeStruct((DD, EE), jnp.float32)
    return pl.pallas_call(
        body,
        grid=(EE // EB,),
        in_specs=[espec, espec, pspec, pspec,
                  w2d, bcol, wdd, brow, bcol,
                  wdd, bcol, wdd, bcol, wdd],
        out_specs=[otsp, otsp],
        out_shape=[osh, osh],
        interpret=interpret,
    )


@functools.lru_cache(maxsize=None)
def _pool_call(interpret=False):
    G = 16

    def body(xT_ref, b_ref, ow_ref, ob_ref, out_ref):
        xT = xT_ref[...]
        b = b_ref[0, :]
        cols = []
        for g in range(G):
            mg = jnp.where((b == g)[None, :], xT, NEGINF)
            cols.append(jnp.max(mg, axis=1))
        a = jnp.stack(cols, axis=1)
        a = jnp.where((a * 0.0) == 0.0, a, 0.0)
        out_ref[...] = _dg(a, ow_ref[...], 0, 0) + ob_ref[...]

    return pl.pallas_call(
        body,
        out_shape=jax.ShapeDtypeStruct((G, 2), jnp.float32),
        interpret=interpret,
    )


@functools.lru_cache(maxsize=None)
def _posdiff_call(interpret=False):
    nchunk = EE // CB
    per_w = (nchunk + NW - 1) // NW

    psh = jax.ShapeDtypeStruct((EE,), jnp.float32)

    @functools.partial(
        pl.kernel,
        out_type=(psh, psh),
        mesh=_mesh(),
        scratch_types=[
            pltpu.VMEM((NN,), jnp.float32),
            pltpu.VMEM((NN,), jnp.float32),
            pltpu.VMEM((CB,), jnp.int32),
            pltpu.VMEM((CB,), jnp.int32),
            pltpu.VMEM((CB,), jnp.float32),
            pltpu.VMEM((CB,), jnp.float32),
        ],
        compiler_params=pltpu.CompilerParams(needs_layout_passes=False),
        interpret=interpret,
    )
    def k(posx_h, posy_h, srcv, dstv, pdx, pdy, posx, posy, sbuf, dbuf, pxb, pyb):
        wid = lax.axis_index("s") * NCORE + lax.axis_index("c")
        pltpu.sync_copy(posx_h, posx)
        pltpu.sync_copy(posy_h, posy)

        def chunk_body(i, _):
            c = wid + NW * i

            @pl.when(c < nchunk)
            def _():
                base = c * CB
                pltpu.sync_copy(srcv.at[pl.ds(base, CB)], sbuf)
                pltpu.sync_copy(dstv.at[pl.ds(base, CB)], dbuf)

                def grp(j, _2):
                    sl = pl.ds(j * 16, 16)
                    s16 = sbuf[sl]
                    d16 = dbuf[sl]
                    pxb[sl] = (plsc.load_gather(posx, [d16])
                               - plsc.load_gather(posx, [s16]))
                    pyb[sl] = (plsc.load_gather(posy, [d16])
                               - plsc.load_gather(posy, [s16]))
                    return 0

                lax.fori_loop(0, CB // 16, grp, 0)
                pltpu.sync_copy(pxb, pdx.at[pl.ds(base, CB)])
                pltpu.sync_copy(pyb, pdy.at[pl.ds(base, CB)])

            return 0

        lax.fori_loop(0, per_w, chunk_body, 0)

    return k


@functools.lru_cache(maxsize=None)
def _gather3_call(interpret=False):
    nblk_total = EP // 1024
    per_w = (nblk_total + NW - 1) // NW
    osh = jax.ShapeDtypeStruct((EP, 2 * DD), jnp.float32)

    @functools.partial(
        pl.kernel,
        out_type=(osh, osh),
        mesh=_mesh(),
        scratch_types=[
            pltpu.VMEM((8, 128), jnp.int32),
            pltpu.VMEM((8, 128), jnp.int32),
            pltpu.VMEM((3, 256, 2 * DD), jnp.float32),
            pltpu.SemaphoreType.DMA,
            pltpu.SemaphoreType.DMA,
            pltpu.SemaphoreType.DMA,
            pltpu.SemaphoreType.DMA,
            pltpu.SemaphoreType.DMA,
            pltpu.SemaphoreType.DMA,
        ],
        interpret=interpret,
    )
    def k(ta, tb, src3, dst3, gsx, gd, sbuf, dbuf, bufs,
          g0, g1, g2, o0, o1, o2):
        wid = lax.axis_index("s") * NCORE + lax.axis_index("c")
        gsem = (g0, g1, g2)
        osem = (o0, o1, o2)

        def blk_body(i, _):
            blk = wid + NW * i

            @pl.when(blk < nblk_total)
            def _():
                pltpu.sync_copy(src3.at[blk], sbuf)
                pltpu.sync_copy(dst3.at[blk], dbuf)
                units = []
                for un in range(8):
                    tbl, ibuf = (tb, dbuf) if un < 4 else (ta, sbuf)
                    q = un % 4
                    out = gd if un < 4 else gsx
                    e0 = blk * 1024 + q * 256
                    units.append((tbl, ibuf, q, out, e0))

                def gstart(un):
                    tbl, ibuf, q, out, e0 = units[un]
                    s = un % 3
                    for j in range(2):
                        pltpu.async_copy(tbl.at[ibuf.at[q * 2 + j]],
                                         bufs.at[s, pl.ds(j * 128, 128)],
                                         gsem[s])

                def gwait(un):
                    tbl, ibuf, q, out, e0 = units[un]
                    s = un % 3
                    for j in range(2):
                        pltpu.make_async_copy(
                            tbl.at[ibuf.at[q * 2 + j]],
                            bufs.at[s, pl.ds(j * 128, 128)], gsem[s]).wait()

                def ostart(un):
                    tbl, ibuf, q, out, e0 = units[un]
                    s = un % 3
                    pltpu.async_copy(bufs.at[s], out.at[pl.ds(e0, 256)], osem[s])

                def owait(un):
                    tbl, ibuf, q, out, e0 = units[un]
                    s = un % 3
                    pltpu.make_async_copy(bufs.at[s], out.at[pl.ds(e0, 256)],
                                          osem[s]).wait()

                gstart(0)
                gstart(1)
                for un in range(8):
                    gwait(un)
                    ostart(un)
                    if un + 2 < 8:
                        if un >= 1:
                            owait(un - 1)
                        gstart(un + 2)
                owait(5)
                owait(6)
                owait(7)

            return 0

        lax.fori_loop(0, per_w, blk_body, 0)

    return k


@functools.lru_cache(maxsize=None)
def _segment_call(interpret=False):
    @functools.partial(
        pl.kernel,
        out_type=jax.ShapeDtypeStruct((DD * NP,), jnp.float32),
        mesh=_mesh(),
        scratch_types=[
            pltpu.VMEM((NP,), jnp.float32),
            pltpu.VMEM((NP,), jnp.float32),
            pltpu.VMEM((NP,), jnp.float32),
            pltpu.VMEM((NP,), jnp.float32),
            pltpu.VMEM((NP,), jnp.float32),
            pltpu.VMEM((NP,), jnp.float32),
            pltpu.VMEM((NP,), jnp.int32),
            pltpu.VMEM((2, CB), jnp.int32),
            pltpu.VMEM((2, CB), jnp.float32),
            pltpu.VMEM((2, CB), jnp.float32),
            pltpu.VMEM((2, CB), jnp.float32),
            pltpu.VMEM((2, CB), jnp.float32),
            pltpu.VMEM((16,), jnp.float32),
            pltpu.VMEM((16,), jnp.int32),
            pltpu.SemaphoreType.DMA,
            pltpu.SemaphoreType.DMA,
        ],
        compiler_params=pltpu.CompilerParams(needs_layout_passes=False),
        interpret=interpret,
    )
    def k(aT, vT, dstv, zeros_c, neginf_c, xoutT,
          m0, m1, s0, s1, t0, t1, di, dbuf, a0b, a1b, v0b, v1b, lf, li,
          semA, semB):
        wid = lax.axis_index("s") * NCORE + lax.axis_index("c")
        c0 = 2 * wid
        c1 = c0 + 1
        iota = lax.iota(jnp.int32, 16)
        sems = (semA, semB)

        pltpu.sync_copy(neginf_c, m0)
        pltpu.sync_copy(neginf_c, m1)
        pltpu.sync_copy(neginf_c, t0)
        pltpu.sync_copy(neginf_c, t1)
        pltpu.sync_copy(zeros_c, s0)
        pltpu.sync_copy(zeros_c, s1)

        def permute(vals, pv):
            lf[...] = vals
            return plsc.load_gather(lf, [pv])

        def shifts_of_keys(sk):
            li[...] = sk
            takes = []
            for kk in (1, 2, 4, 8):
                skk = plsc.load_gather(li, [jnp.maximum(iota - kk, 0)])
                takes.append((sk == skk) & (iota >= kk))
            sku = plsc.load_gather(li, [jnp.minimum(iota + 1, 15)])
            ml = (sk != sku) | (iota == 15)
            return takes, ml

        def seg_scan(vals, takes, is_sum):
            v = vals
            for kk, take in zip((1, 2, 4, 8), takes):
                lf[...] = v
                sh = plsc.load_gather(lf, [jnp.maximum(iota - kk, 0)])
                if is_sum:
                    v = v + jnp.where(take, sh, 0.0)
                else:
                    v = jnp.where(take, jnp.maximum(v, sh), v)
            return v

        def rmw_max(acc, idxv, vals, mask=None):
            cur = plsc.load_gather(acc, [idxv], mask=mask)
            plsc.store_scatter(acc, [idxv], jnp.maximum(cur, vals), mask=mask)

        def rmw_add(acc, idxv, vals, mask=None):
            cur = plsc.load_gather(acc, [idxv], mask=mask)
            plsc.store_scatter(acc, [idxv], cur + vals, mask=mask)

        nch = EE // CB

        def detect(d16):
            plsc.store_scatter(di, [d16], iota)
            rb = plsc.load_gather(di, [d16])
            return rb != iota

        def p1_copies(slot, i):
            base = i * CB
            return [
                pltpu.make_async_copy(dstv.at[pl.ds(base, CB)],
                                      dbuf.at[slot], sems[slot]),
                pltpu.make_async_copy(
                    aT.at[pl.ds(pl.multiple_of(c0 * EE + base, 128), CB)],
                    a0b.at[slot], sems[slot]),
                pltpu.make_async_copy(
                    aT.at[pl.ds(pl.multiple_of(c1 * EE + base, 128), CB)],
                    a1b.at[slot], sems[slot]),
            ]

        def p1_start(slot, i):
            for d in p1_copies(slot, i):
                d.start()

        def p1_wait(slot, i):
            for d in p1_copies(slot, i):
                d.wait()

        def p1_proc(slot):
            def batch(jb, _):
                q0 = jb * 8
                d16s, a0s, a1s = [], [], []
                mism = None
                for u in range(8):
                    sl = pl.ds((q0 + u) * 16, 16)
                    d16 = dbuf[slot, sl]
                    mm = detect(d16)
                    mism = mm if mism is None else (mism | mm)
                    d16s.append(d16)
                    a0s.append(a0b[slot, sl])
                    a1s.append(a1b[slot, sl])

                def slow():
                    for u in range(8):
                        sk, pv = plsc.sort_key_val(d16s[u], iota)
                        takes, ml = shifts_of_keys(sk)
                        rmw_max(m0, sk, seg_scan(permute(a0s[u], pv), takes, False), mask=ml)
                        rmw_max(m1, sk, seg_scan(permute(a1s[u], pv), takes, False), mask=ml)

                def fast():
                    for u in range(8):
                        rmw_max(m0, d16s[u], a0s[u])
                        rmw_max(m1, d16s[u], a1s[u])

                lax.cond(jnp.any(mism), slow, fast)
                return 0

            lax.fori_loop(0, CB // 128, batch, 0)

        p1_start(0, 0)

        def p1_pair(ip, _):
            i0 = 2 * ip
            p1_wait(0, i0)
            p1_start(1, i0 + 1)
            p1_proc(0)
            p1_wait(1, i0 + 1)

            @pl.when(i0 + 2 < nch)
            def _():
                p1_start(0, i0 + 2)

            p1_proc(1)
            return 0

        lax.fori_loop(0, nch // 2, p1_pair, 0)

        def p2_copies(slot, i):
            base = i * CB
            return [
                pltpu.make_async_copy(dstv.at[pl.ds(base, CB)],
                                      dbuf.at[slot], sems[slot]),
                pltpu.make_async_copy(
                    aT.at[pl.ds(pl.multiple_of(c0 * EE + base, 128), CB)],
                    a0b.at[slot], sems[slot]),
                pltpu.make_async_copy(
                    aT.at[pl.ds(pl.multiple_of(c1 * EE + base, 128), CB)],
                    a1b.at[slot], sems[slot]),
                pltpu.make_async_copy(
                    vT.at[pl.ds(pl.multiple_of(c0 * EE + base, 128), CB)],
                    v0b.at[slot], sems[slot]),
                pltpu.make_async_copy(
                    vT.at[pl.ds(pl.multiple_of(c1 * EE + base, 128), CB)],
                    v1b.at[slot], sems[slot]),
            ]

        def p2_start(slot, i):
            for d in p2_copies(slot, i):
                d.start()

        def p2_wait(slot, i):
            for d in p2_copies(slot, i):
                d.wait()

        def p2_proc(slot):
            def batch(jb, _):
                q0 = jb * 4
                d16s, e0s, e1s, p0s, p1s = [], [], [], [], []
                mism = None
                for u in range(4):
                    sl = pl.ds((q0 + u) * 16, 16)
                    d16 = dbuf[slot, sl]
                    mm = detect(d16)
                    mism = mm if mism is None else (mism | mm)
                    e0 = jnp.exp(a0b[slot, sl] - plsc.load_gather(m0, [d16]))
                    e1 = jnp.exp(a1b[slot, sl] - plsc.load_gather(m1, [d16]))
                    d16s.append(d16)
                    e0s.append(e0)
                    e1s.append(e1)
                    p0s.append(e0 * v0b[slot, sl])
                    p1s.append(e1 * v1b[slot, sl])

                def slow():
                    for u in range(4):
                        sk, pv = plsc.sort_key_val(d16s[u], iota)
                        takes, ml = shifts_of_keys(sk)
                        rmw_add(s0, sk, seg_scan(permute(e0s[u], pv), takes, True), mask=ml)
                        rmw_add(s1, sk, seg_scan(permute(e1s[u], pv), takes, True), mask=ml)
                        rmw_max(t0, sk, seg_scan(permute(p0s[u], pv), takes, False), mask=ml)
                        rmw_max(t1, sk, seg_scan(permute(p1s[u], pv), takes, False), mask=ml)

                def fast():
                    for u in range(4):
                        plsc.addupdate_scatter(s0, [d16s[u]], e0s[u])
                        plsc.addupdate_scatter(s1, [d16s[u]], e1s[u])
                        rmw_max(t0, d16s[u], p0s[u])
                        rmw_max(t1, d16s[u], p1s[u])

                lax.cond(jnp.any(mism), slow, fast)
                return 0

            lax.fori_loop(0, CB // 64, batch, 0)

        p2_start(0, 0)

        def p2_pair(ip, _):
            i0 = 2 * ip
            p2_wait(0, i0)
            p2_start(1, i0 + 1)
            p2_proc(0)
            p2_wait(1, i0 + 1)

            @pl.when(i0 + 2 < nch)
            def _():
                p2_start(0, i0 + 2)

            p2_proc(1)
            return 0

        lax.fori_loop(0, nch // 2, p2_pair, 0)

        def ep_blk(i, _):
            base = i * CB

            def grp(j, _2):
                sl = pl.ds(base + j * 16, 16)
                osl = pl.ds(j * 16, 16)
                tv0 = t0[sl]
                tv1 = t1[sl]
                sv0 = s0[sl]
                sv1 = s1[sl]
                a0b[0, osl] = jnp.where((tv0 * 0.0) == 0.0, tv0 / (sv0 + 1e-16), 0.0)
                a1b[0, osl] = jnp.where((tv1 * 0.0) == 0.0, tv1 / (sv1 + 1e-16), 0.0)
                return 0

            lax.fori_loop(0, CB // 16, grp, 0)
            pltpu.sync_copy(a0b.at[0], xoutT.at[pl.ds(pl.multiple_of(c0 * NP + base, 128), CB)])
            pltpu.sync_copy(a1b.at[0], xoutT.at[pl.ds(pl.multiple_of(c1 * NP + base, 128), CB)])
            return 0

        lax.fori_loop(0, NP // CB, ep_blk, 0)

    return k


def _run(x_clusters, pos_clusters, edge_index_clusters, batch,
         lin_w, lin_b, src_w, src_b, dst_w, dst_b,
         pos_w1, pos_b1, pos_w2, pos_b2,
         attn_w1, attn_b1, attn_w2, attn_b2,
         out_w, out_b, interpret=False):
    f32 = jnp.float32
    src = edge_index_clusters[0]
    dst = edge_index_clusters[1]
    pad_e = EP - EE
    src3 = jnp.pad(src, (0, pad_e)).reshape(EP // 1024, 8, 128)
    dst3 = jnp.pad(dst, (0, pad_e)).reshape(EP // 1024, 8, 128)
    posx = pos_clusters[:, 0] + 0.0
    posy = pos_clusters[:, 1] + 0.0
    eye = jnp.eye(DD, dtype=f32)
    zeros_c = jnp.zeros((NP,), f32)
    neginf_c = jnp.full((NP,), NEGINF, f32)
    batch_p = jnp.pad(batch, (0, NP - NN), constant_values=16).reshape(1, NP)

    pdx, pdy = _posdiff_call(interpret)(posx, posy, src, dst)
    pdx = pdx.reshape(1, EE)
    pdy = pdy.reshape(1, EE)

    xt = jnp.pad(x_clusters, ((0, NP - NN), (0, 0)))
    nlayers = lin_w.shape[0]
    for i in range(nlayers):
        ta, tb = _node_tables_call(i > 0, interpret)(
            xt, lin_w[i], lin_b[i].reshape(1, DD),
            src_w[i], src_b[i].reshape(1, DD),
            dst_w[i], dst_b[i].reshape(1, DD))
        gsx, gd = _gather3_call(interpret)(ta, tb, src3, dst3)
        aT, vT = _edge_call(interpret)(
            gd, gsx, pdx, pdy,
            pos_w1[i], pos_b1[i].reshape(DD, 1),
            pos_w2[i], pos_b2[i].reshape(1, DD), pos_b2[i].reshape(DD, 1),
            attn_w1[i], attn_b1[i].reshape(DD, 1),
            attn_w2[i], attn_b2[i].reshape(DD, 1),
            eye)
        xt1 = _segment_call(interpret)(aT.reshape(DD * EE), vT.reshape(DD * EE),
                                       dst, zeros_c, neginf_c)
        xt = xt1.reshape(DD, NP)

    return _pool_call(interpret)(xt, batch_p, out_w, out_b.reshape(1, 2))


@jax.jit
def kernel(x_clusters, pos_clusters, edge_index_clusters, batch, add_cluster_pos,
           lin_w, lin_b, src_w, src_b, dst_w, dst_b,
           pos_w1, pos_b1, pos_w2, pos_b2,
           attn_w1, attn_b1, attn_w2, attn_b2,
           out_w, out_b):
    del add_cluster_pos
    return _run(x_clusters, pos_clusters, edge_index_clusters, batch,
                lin_w, lin_b, src_w, src_b, dst_w, dst_b,
                pos_w1, pos_b1, pos_w2, pos_b2,
                attn_w1, attn_b1, attn_w2, attn_b2,
                out_w, out_b)
